# baseline (speedup 1.0000x reference)
import functools

import jax
import jax.numpy as jnp
from jax import lax
from jax.experimental import pallas as pl
from jax.experimental.pallas import tpu as pltpu
from jax.experimental.pallas import tpu_sc as plsc

N = 10000
F = 128
E = 320000
NC = 2
NS = 16
NW = NC * NS
CHUNK = 128
NCH = 79
EPT = NCH * CHUNK
E_PAD = NW * EPT
NDEG = 10112
DEG_SL = NDEG // NS
NACC = 10240
ACC_SL = NACC // NS
HF = F // NC
NCHT = NCH * 2
BLK = 1024
GRID = (N + BLK - 1) // BLK

_mesh = plsc.VectorSubcoreMesh(
    core_axis_name="c", subcore_axis_name="s", num_cores=NC, num_subcores=NS)


def _deg_body(dst_hbm, ones_hbm, zeros_hbm, out_hbm, idx_v, ones_v, zbuf_v,
              deg_sh):
    cid = lax.axis_index("c")
    sid = lax.axis_index("s")
    wid = sid * NC + cid
    pltpu.sync_copy(zeros_hbm, zbuf_v)
    pltpu.sync_copy(zbuf_v, deg_sh.at[pl.ds(sid * DEG_SL, DEG_SL)])
    pltpu.sync_copy(ones_hbm, ones_v)
    pltpu.sync_copy(dst_hbm.at[wid], idx_v)
    plsc.subcore_barrier()

    def body(j, carry):
        pltpu.sync_copy(ones_v, deg_sh.at[idx_v.at[j]], add=True)
        return carry

    lax.fori_loop(0, NCH, body, 0)
    plsc.subcore_barrier()
    pltpu.sync_copy(deg_sh.at[pl.ds(sid * DEG_SL, DEG_SL)], zbuf_v)
    pltpu.sync_copy(zbuf_v,
                    out_hbm.at[pl.ds(cid * NDEG + sid * DEG_SL, DEG_SL)])


_deg_call = functools.partial(
    pl.kernel,
    out_type=jax.ShapeDtypeStruct((NC * NDEG,), jnp.float32),
    mesh=_mesh,
    scratch_types=[
        pltpu.VMEM((NCH, CHUNK), jnp.int32),
        pltpu.VMEM((CHUNK,), jnp.float32),
        pltpu.VMEM((DEG_SL,), jnp.float32),
        pltpu.VMEM_SHARED((NDEG,), jnp.float32),
    ],
)(_deg_body)


def _msg_body(hpL_hbm, hpR_hbm, src_hbm, dst_hbm, zrows_hbm, out_hbm,
              src_v, dst_v, mbuf, sem, sem_s, acc_sh):
    cid = lax.axis_index("c")
    sid = lax.axis_index("s")
    pltpu.sync_copy(zrows_hbm, mbuf.at[0])
    for k in range(ACC_SL // CHUNK):
        pltpu.sync_copy(mbuf.at[0],
                        acc_sh.at[pl.ds(sid * ACC_SL + k * CHUNK, CHUNK)])
    pltpu.sync_copy(src_hbm.at[2 * sid], src_v.at[pl.ds(0, NCH)])
    pltpu.sync_copy(src_hbm.at[2 * sid + 1], src_v.at[pl.ds(NCH, NCH)])
    pltpu.sync_copy(dst_hbm.at[2 * sid], dst_v.at[pl.ds(0, NCH)])
    pltpu.sync_copy(dst_hbm.at[2 * sid + 1], dst_v.at[pl.ds(NCH, NCH)])
    plsc.subcore_barrier()

    def run(hp_hbm):
        pltpu.async_copy(hp_hbm.at[src_v.at[0]], mbuf.at[0], sem)

        def body(j, carry):
            jb = lax.rem(j, 2)
            nb = lax.rem(j + 1, 2)
            pltpu.make_async_copy(hp_hbm.at[src_v.at[j]], mbuf.at[jb],
                                  sem).wait()

            @pl.when(j >= 1)
            def _():
                pltpu.make_async_copy(mbuf.at[nb],
                                      acc_sh.at[dst_v.at[j - 1]],
                                      sem_s).wait()

            pltpu.async_copy(mbuf.at[jb], acc_sh.at[dst_v.at[j]], sem_s,
                             add=True)

            @pl.when(j + 1 < NCHT)
            def _():
                pltpu.async_copy(hp_hbm.at[src_v.at[j + 1]], mbuf.at[nb],
                                 sem)

            return carry

        lax.fori_loop(0, NCHT, body, 0)
        pltpu.make_async_copy(mbuf.at[lax.rem(NCHT - 1, 2)],
                              acc_sh.at[dst_v.at[NCHT - 1]], sem_s).wait()

    @pl.when(cid == 0)
    def _():
        run(hpL_hbm)

    @pl.when(cid == 1)
    def _():
        run(hpR_hbm)

    plsc.subcore_barrier()
    for k in range(ACC_SL // CHUNK):
        kb = k % 2
        base = sid * ACC_SL + k * CHUNK
        pltpu.sync_copy(acc_sh.at[pl.ds(base, CHUNK)], mbuf.at[kb])
        pltpu.sync_copy(mbuf.at[kb], out_hbm.at[cid, pl.ds(base, CHUNK)])


_msg_call = functools.partial(
    pl.kernel,
    out_type=jax.ShapeDtypeStruct((NC, NACC, HF), jnp.float32),
    mesh=_mesh,
    scratch_types=[
        pltpu.VMEM((NCHT, CHUNK), jnp.int32),
        pltpu.VMEM((NCHT, CHUNK), jnp.int32),
        pltpu.VMEM((2, CHUNK, HF), jnp.float32),
        pltpu.SemaphoreType.DMA,
        pltpu.SemaphoreType.DMA,
        pltpu.VMEM_SHARED((NACC, HF), jnp.float32),
    ],
    compiler_params=pltpu.CompilerParams(use_tc_tiling_on_sc=False),
)(_msg_body)


def _tc1_body(x_ref, w_ref, degp_ref, hpL_ref, hpR_ref, dinv_ref):
    deg = degp_ref[0, :] + degp_ref[1, :] + 1.0
    dinv = lax.rsqrt(deg)
    h = jnp.dot(x_ref[...], w_ref[...], preferred_element_type=jnp.float32)
    hp = h * dinv[:, None]
    hpL_ref[...] = hp[:, :HF]
    hpR_ref[...] = hp[:, HF:]
    dinv_ref[...] = dinv[:, None]


def _tc1_call(x, W_gc, degp):
    return pl.pallas_call(
        _tc1_body,
        grid=(GRID,),
        in_specs=[
            pl.BlockSpec((BLK, F), lambda i: (i, 0)),
            pl.BlockSpec((F, F), lambda i: (0, 0)),
            pl.BlockSpec((NC, BLK), lambda i: (0, i)),
        ],
        out_specs=[
            pl.BlockSpec((BLK, HF), lambda i: (i, 0)),
            pl.BlockSpec((BLK, HF), lambda i: (i, 0)),
            pl.BlockSpec((BLK, 1), lambda i: (i, 0)),
        ],
        out_shape=[
            jax.ShapeDtypeStruct((N, HF), jnp.float32),
            jax.ShapeDtypeStruct((N, HF), jnp.float32),
            jax.ShapeDtypeStruct((N, 1), jnp.float32),
        ],
    )(x, W_gc, degp)


def _tc2_body(accp_ref, hpL_ref, hpR_ref, dinv_ref, t_ref, bgc_ref,
              w00_ref, b00_ref, w10_ref, b10_ref,
              w01_ref, b01_ref, w11_ref, b11_ref, wps_ref, bps_ref,
              y_ref, dist_ref, ps_ref):
    acc = jnp.concatenate(
        [accp_ref[0] + hpL_ref[...], accp_ref[1] + hpR_ref[...]], axis=1)
    dist = jnp.maximum(acc * dinv_ref[...] + bgc_ref[...], 0.0)
    dist_ref[...] = dist
    y00 = jnp.maximum(
        jnp.dot(dist, w00_ref[...], preferred_element_type=jnp.float32)
        + b00_ref[...], 0.0)
    y0 = jnp.dot(y00, w01_ref[...], preferred_element_type=jnp.float32) \
        + b01_ref[...]
    y10 = jnp.maximum(
        jnp.dot(dist, w10_ref[...], preferred_element_type=jnp.float32)
        + b10_ref[...], 0.0)
    y1 = jnp.dot(y10, w11_ref[...], preferred_element_type=jnp.float32) \
        + b11_ref[...]
    y_ref[...] = jnp.where(t_ref[...] > 0, y1, y0)
    ps = jnp.dot(dist, wps_ref[...], preferred_element_type=jnp.float32) \
        + bps_ref[...]
    ps_ref[...] = jax.nn.sigmoid(ps)


def _tc2_call(accp, hpL, hpR, dinv, t2, bgc, w00, b00, w10, b10,
              w01, b01, w11, b11, wps, bps):
    def full(shape):
        nd = len(shape)
        return pl.BlockSpec(shape, lambda i, _nd=nd: (0,) * _nd)
    return pl.pallas_call(
        _tc2_body,
        grid=(GRID,),
        in_specs=[
            pl.BlockSpec((NC, BLK, HF), lambda i: (0, i, 0)),
            pl.BlockSpec((BLK, HF), lambda i: (i, 0)),
            pl.BlockSpec((BLK, HF), lambda i: (i, 0)),
            pl.BlockSpec((BLK, 1), lambda i: (i, 0)),
            pl.BlockSpec((BLK, 1), lambda i: (i, 0)),
            full((1, F)),
            full((F, F)), full((1, F)),
            full((F, F)), full((1, F)),
            full((F, 1)), full((1, 1)),
            full((F, 1)), full((1, 1)),
            full((F, 1)), full((1, 1)),
        ],
        out_specs=[
            pl.BlockSpec((BLK, 1), lambda i: (i, 0)),
            pl.BlockSpec((BLK, F), lambda i: (i, 0)),
            pl.BlockSpec((BLK, 1), lambda i: (i, 0)),
        ],
        out_shape=[
            jax.ShapeDtypeStruct((N, 1), jnp.float32),
            jax.ShapeDtypeStruct((N, F), jnp.float32),
            jax.ShapeDtypeStruct((N, 1), jnp.float32),
        ],
    )(accp, hpL, hpR, dinv, t2, bgc, w00, b00, w10, b10,
      w01, b01, w11, b11, wps, bps)


def kernel(x, edge_index, t, W_gc, b_gc, W_t00, b_t00, W_t10, b_t10,
           W_t01, b_t01, W_t11, b_t11, W_ps, b_ps):
    src = edge_index[0]
    dst = edge_index[1]
    npad = E_PAD - E
    ar = jnp.arange(npad, dtype=jnp.int32)
    src_p = jnp.concatenate([src, (ar * 7919) % N])
    dst_p = jnp.concatenate([dst, N + (ar % 64)])
    src3 = src_p.reshape(NW, NCH, CHUNK)
    dst3 = dst_p.reshape(NW, NCH, CHUNK)

    ones = jnp.ones((CHUNK,), jnp.float32)
    zeros1 = jnp.zeros((DEG_SL,), jnp.float32)
    zrows = jnp.zeros((CHUNK, HF), jnp.float32)

    degp = _deg_call(dst3, ones, zeros1).reshape(NC, NDEG)
    hpL, hpR, dinv = _tc1_call(x, W_gc, degp)
    accp = _msg_call(hpL, hpR, src3, dst3, zrows)
    y, dist, ps = _tc2_call(
        accp, hpL, hpR, dinv, t.reshape(N, 1),
        b_gc.reshape(1, F), W_t00, b_t00.reshape(1, F),
        W_t10, b_t10.reshape(1, F), W_t01, b_t01.reshape(1, 1),
        W_t11, b_t11.reshape(1, 1), W_ps, b_ps.reshape(1, 1))
    return (y.reshape(-1), dist, ps.reshape(-1))


# speedup vs baseline: 1.1994x; 1.1994x over previous
import functools

import jax
import jax.numpy as jnp
from jax import lax
from jax.experimental import pallas as pl
from jax.experimental.pallas import tpu as pltpu
from jax.experimental.pallas import tpu_sc as plsc

N = 10000
F = 128
E = 320000
NC = 2
NS = 16
NW = NC * NS
CHUNK = 128
NCH = 80
EPT = NCH * CHUNK
E_PAD = NW * EPT
NROW = E_PAD // CHUNK
NDEG = 10112
DEG_SL = NDEG // NS
NACC = 10240
ACC_SL = NACC // NS
HF = F // NC
NCHT = NCH * 2
KR = 2
NGRP = NCHT // KR
GROW = KR * CHUNK
BLK = 1024
GRID = (N + BLK - 1) // BLK

_mesh = plsc.VectorSubcoreMesh(
    core_axis_name="c", subcore_axis_name="s", num_cores=NC, num_subcores=NS)


def _deg_body(dst_hbm, ones_hbm, zeros_hbm, out_hbm, idx_v, ones_v, zbuf_v,
              deg_sh):
    cid = lax.axis_index("c")
    sid = lax.axis_index("s")
    wid = sid * NC + cid
    pltpu.sync_copy(zeros_hbm, zbuf_v)
    pltpu.sync_copy(zbuf_v, deg_sh.at[pl.ds(sid * DEG_SL, DEG_SL)])
    pltpu.sync_copy(ones_hbm, ones_v)
    pltpu.sync_copy(dst_hbm.at[pl.ds(wid * NCH, NCH)], idx_v)
    plsc.subcore_barrier()

    def body(j, carry):
        pltpu.sync_copy(ones_v, deg_sh.at[idx_v.at[j]], add=True)
        return carry

    lax.fori_loop(0, NCH, body, 0)
    plsc.subcore_barrier()
    pltpu.sync_copy(deg_sh.at[pl.ds(sid * DEG_SL, DEG_SL)], zbuf_v)
    pltpu.sync_copy(zbuf_v,
                    out_hbm.at[pl.ds(cid * NDEG + sid * DEG_SL, DEG_SL)])


_deg_call = functools.partial(
    pl.kernel,
    out_type=jax.ShapeDtypeStruct((NC * NDEG,), jnp.float32),
    mesh=_mesh,
    scratch_types=[
        pltpu.VMEM((NCH, CHUNK), jnp.int32),
        pltpu.VMEM((CHUNK,), jnp.float32),
        pltpu.VMEM((DEG_SL,), jnp.float32),
        pltpu.VMEM_SHARED((NDEG,), jnp.float32),
    ],
)(_deg_body)


def _msg_body(hpL_hbm, hpR_hbm, src_hbm, dst_hbm, zrows_hbm, out_hbm,
              src_v, dst_v, gbuf, sem, sem_s, acc_sh):
    cid = lax.axis_index("c")
    sid = lax.axis_index("s")
    pltpu.sync_copy(zrows_hbm, gbuf.at[0, 0])
    for k in range(ACC_SL // CHUNK):
        pltpu.sync_copy(gbuf.at[0, 0],
                        acc_sh.at[pl.ds(sid * ACC_SL + k * CHUNK, CHUNK)])
    pltpu.sync_copy(src_hbm.at[pl.ds(sid * NCHT, NCHT)], src_v)
    pltpu.sync_copy(dst_hbm.at[pl.ds(sid * NCHT, NCHT)], dst_v)
    plsc.subcore_barrier()

    def run(hp_hbm):
        for k in range(KR):
            pltpu.async_copy(hp_hbm.at[src_v.at[k]], gbuf.at[0, k], sem)

        def body(g, carry):
            gb = lax.rem(g, 2)
            nb = lax.rem(g + 1, 2)
            base = g * KR
            for k in range(KR):
                pltpu.make_async_copy(
                    hp_hbm.at[src_v.at[base + k]],
                    gbuf.at[gb, k], sem).wait()

            @pl.when(g >= 1)
            def _():
                for k in range(KR):
                    pltpu.make_async_copy(
                        gbuf.at[nb, k],
                        acc_sh.at[dst_v.at[(g - 1) * KR + k]], sem_s).wait()

            @pl.when(g + 1 < NGRP)
            def _():
                for k in range(KR):
                    pltpu.async_copy(
                        hp_hbm.at[src_v.at[(g + 1) * KR + k]],
                        gbuf.at[nb, k], sem)

            for k in range(KR):
                pltpu.async_copy(gbuf.at[gb, k],
                                 acc_sh.at[dst_v.at[base + k]], sem_s,
                                 add=True)
            return carry

        lax.fori_loop(0, NGRP, body, 0)
        for k in range(KR):
            pltpu.make_async_copy(
                gbuf.at[lax.rem(NGRP - 1, 2), k],
                acc_sh.at[dst_v.at[(NGRP - 1) * KR + k]], sem_s).wait()

    @pl.when(cid == 0)
    def _():
        run(hpL_hbm)

    @pl.when(cid == 1)
    def _():
        run(hpR_hbm)

    plsc.subcore_barrier()
    for k in range(ACC_SL // CHUNK):
        kb = k % 2
        base = sid * ACC_SL + k * CHUNK
        pltpu.sync_copy(acc_sh.at[pl.ds(base, CHUNK)], gbuf.at[0, kb])
        pltpu.sync_copy(gbuf.at[0, kb], out_hbm.at[cid, pl.ds(base, CHUNK)])


_msg_call = functools.partial(
    pl.kernel,
    out_type=jax.ShapeDtypeStruct((NC, NACC, HF), jnp.float32),
    mesh=_mesh,
    scratch_types=[
        pltpu.VMEM((NCHT, CHUNK), jnp.int32),
        pltpu.VMEM((NCHT, CHUNK), jnp.int32),
        pltpu.VMEM((2, KR, CHUNK, HF), jnp.float32),
        pltpu.SemaphoreType.DMA,
        pltpu.SemaphoreType.DMA,
        pltpu.VMEM_SHARED((NACC, HF), jnp.float32),
    ],
    compiler_params=pltpu.CompilerParams(use_tc_tiling_on_sc=False),
)(_msg_body)


def _tc1_body(x_ref, w_ref, degp_ref, hpL_ref, hpR_ref, dinv_ref):
    deg = degp_ref[0, :] + degp_ref[1, :] + 1.0
    dinv = lax.rsqrt(deg)
    h = jnp.dot(x_ref[...], w_ref[...], preferred_element_type=jnp.float32)
    hp = h * dinv[:, None]
    hpL_ref[...] = hp[:, :HF]
    hpR_ref[...] = hp[:, HF:]
    dinv_ref[...] = dinv[:, None]


def _tc1_call(x, W_gc, degp):
    return pl.pallas_call(
        _tc1_body,
        grid=(GRID,),
        in_specs=[
            pl.BlockSpec((BLK, F), lambda i: (i, 0)),
            pl.BlockSpec((F, F), lambda i: (0, 0)),
            pl.BlockSpec((NC, BLK), lambda i: (0, i)),
        ],
        out_specs=[
            pl.BlockSpec((BLK, HF), lambda i: (i, 0)),
            pl.BlockSpec((BLK, HF), lambda i: (i, 0)),
            pl.BlockSpec((BLK, 1), lambda i: (i, 0)),
        ],
        out_shape=[
            jax.ShapeDtypeStruct((N, HF), jnp.float32),
            jax.ShapeDtypeStruct((N, HF), jnp.float32),
            jax.ShapeDtypeStruct((N, 1), jnp.float32),
        ],
    )(x, W_gc, degp)


def _tc2_body(accp_ref, hpL_ref, hpR_ref, dinv_ref, t_ref, bgc_ref,
              w00_ref, b00_ref, w10_ref, b10_ref,
              w01_ref, b01_ref, w11_ref, b11_ref, wps_ref, bps_ref,
              y_ref, dist_ref, ps_ref):
    acc = jnp.concatenate(
        [accp_ref[0] + hpL_ref[...], accp_ref[1] + hpR_ref[...]], axis=1)
    dist = jnp.maximum(acc * dinv_ref[...] + bgc_ref[...], 0.0)
    dist_ref[...] = dist
    y00 = jnp.maximum(
        jnp.dot(dist, w00_ref[...], preferred_element_type=jnp.float32)
        + b00_ref[...], 0.0)
    y0 = jnp.dot(y00, w01_ref[...], preferred_element_type=jnp.float32) \
        + b01_ref[...]
    y10 = jnp.maximum(
        jnp.dot(dist, w10_ref[...], preferred_element_type=jnp.float32)
        + b10_ref[...], 0.0)
    y1 = jnp.dot(y10, w11_ref[...], preferred_element_type=jnp.float32) \
        + b11_ref[...]
    y_ref[...] = jnp.where(t_ref[...] > 0, y1, y0)
    ps = jnp.dot(dist, wps_ref[...], preferred_element_type=jnp.float32) \
        + bps_ref[...]
    ps_ref[...] = jax.nn.sigmoid(ps)


def _tc2_call(accp, hpL, hpR, dinv, t2, bgc, w00, b00, w10, b10,
              w01, b01, w11, b11, wps, bps):
    def full(shape):
        nd = len(shape)
        return pl.BlockSpec(shape, lambda i, _nd=nd: (0,) * _nd)
    return pl.pallas_call(
        _tc2_body,
        grid=(GRID,),
        in_specs=[
            pl.BlockSpec((NC, BLK, HF), lambda i: (0, i, 0)),
            pl.BlockSpec((BLK, HF), lambda i: (i, 0)),
            pl.BlockSpec((BLK, HF), lambda i: (i, 0)),
            pl.BlockSpec((BLK, 1), lambda i: (i, 0)),
            pl.BlockSpec((BLK, 1), lambda i: (i, 0)),
            full((1, F)),
            full((F, F)), full((1, F)),
            full((F, F)), full((1, F)),
            full((F, 1)), full((1, 1)),
            full((F, 1)), full((1, 1)),
            full((F, 1)), full((1, 1)),
        ],
        out_specs=[
            pl.BlockSpec((BLK, 1), lambda i: (i, 0)),
            pl.BlockSpec((BLK, F), lambda i: (i, 0)),
            pl.BlockSpec((BLK, 1), lambda i: (i, 0)),
        ],
        out_shape=[
            jax.ShapeDtypeStruct((N, 1), jnp.float32),
            jax.ShapeDtypeStruct((N, F), jnp.float32),
            jax.ShapeDtypeStruct((N, 1), jnp.float32),
        ],
    )(accp, hpL, hpR, dinv, t2, bgc, w00, b00, w10, b10,
      w01, b01, w11, b11, wps, bps)


def kernel(x, edge_index, t, W_gc, b_gc, W_t00, b_t00, W_t10, b_t10,
           W_t01, b_t01, W_t11, b_t11, W_ps, b_ps):
    src = edge_index[0]
    dst = edge_index[1]
    npad = E_PAD - E
    ar = jnp.arange(npad, dtype=jnp.int32)
    src_p = jnp.concatenate([src, (ar * 7919) % N])
    dst_p = jnp.concatenate([dst, N + (ar % 64)])
    src3 = src_p.reshape(NROW, CHUNK)
    dst3 = dst_p.reshape(NROW, CHUNK)

    ones = jnp.ones((CHUNK,), jnp.float32)
    zeros1 = jnp.zeros((DEG_SL,), jnp.float32)
    zrows = jnp.zeros((CHUNK, HF), jnp.float32)

    degp = _deg_call(dst3, ones, zeros1).reshape(NC, NDEG)
    hpL, hpR, dinv = _tc1_call(x, W_gc, degp)
    accp = _msg_call(hpL, hpR, src3, dst3, zrows)
    y, dist, ps = _tc2_call(
        accp, hpL, hpR, dinv, t.reshape(N, 1),
        b_gc.reshape(1, F), W_t00, b_t00.reshape(1, F),
        W_t10, b_t10.reshape(1, F), W_t01, b_t01.reshape(1, 1),
        W_t11, b_t11.reshape(1, 1), W_ps, b_ps.reshape(1, 1))
    return (y.reshape(-1), dist, ps.reshape(-1))


# speedup vs baseline: 1.3913x; 1.1600x over previous
import functools

import jax
import jax.numpy as jnp
from jax import lax
from jax.experimental import pallas as pl
from jax.experimental.pallas import tpu as pltpu
from jax.experimental.pallas import tpu_sc as plsc

N = 10000
F = 128
E = 320000
NC = 2
NS = 16
NW = NC * NS
CHUNK = 128
NCH = 80
EPT = NCH * CHUNK
E_PAD = NW * EPT
NROW = E_PAD // CHUNK
NDEG = 10112
DEG_SL = NDEG // NS
NACC = 10240
ACC_SL = NACC // NS
HF = F // NC
NCHT = NCH * 2
KR = 2
NGRP = NCHT // KR
GROW = KR * CHUNK
BLK = 1024
GRID = (N + BLK - 1) // BLK

_mesh = plsc.VectorSubcoreMesh(
    core_axis_name="c", subcore_axis_name="s", num_cores=NC, num_subcores=NS)


def _deg_body(dst_hbm, ones_hbm, zeros_hbm, out_hbm, idx_v, ones_v, zbuf_v,
              deg_sh):
    cid = lax.axis_index("c")
    sid = lax.axis_index("s")
    wid = sid * NC + cid
    pltpu.sync_copy(zeros_hbm, zbuf_v)
    pltpu.sync_copy(zbuf_v, deg_sh.at[pl.ds(sid * DEG_SL, DEG_SL)])
    pltpu.sync_copy(ones_hbm, ones_v)
    pltpu.sync_copy(dst_hbm.at[pl.ds(wid * NCH, NCH)], idx_v)
    plsc.subcore_barrier()

    def body(j, carry):
        pltpu.sync_copy(ones_v, deg_sh.at[idx_v.at[j]], add=True)
        return carry

    lax.fori_loop(0, NCH, body, 0)
    plsc.subcore_barrier()
    pltpu.sync_copy(deg_sh.at[pl.ds(sid * DEG_SL, DEG_SL)], zbuf_v)
    pltpu.sync_copy(zbuf_v,
                    out_hbm.at[pl.ds(cid * NDEG + sid * DEG_SL, DEG_SL)])


_deg_call = functools.partial(
    pl.kernel,
    out_type=jax.ShapeDtypeStruct((NC * NDEG,), jnp.float32),
    mesh=_mesh,
    scratch_types=[
        pltpu.VMEM((NCH, CHUNK), jnp.int32),
        pltpu.VMEM((CHUNK,), jnp.float32),
        pltpu.VMEM((DEG_SL,), jnp.float32),
        pltpu.VMEM_SHARED((NDEG,), jnp.float32),
    ],
)(_deg_body)


def _msg_body(hpL_hbm, hpR_hbm, src_hbm, dst_hbm, zrows_hbm, out_hbm,
              src_v, dst_v, gbuf, sem, sem_s, acc_sh):
    cid = lax.axis_index("c")
    sid = lax.axis_index("s")
    pltpu.sync_copy(zrows_hbm, gbuf.at[0, 0])
    for k in range(ACC_SL // CHUNK):
        pltpu.sync_copy(gbuf.at[0, 0],
                        acc_sh.at[pl.ds(sid * ACC_SL + k * CHUNK, CHUNK)])
    pltpu.sync_copy(src_hbm.at[pl.ds(sid * NCHT, NCHT)], src_v)
    pltpu.sync_copy(dst_hbm.at[pl.ds(sid * NCHT, NCHT)], dst_v)
    plsc.subcore_barrier()

    def run(hp_hbm):
        for k in range(KR):
            pltpu.async_copy(hp_hbm.at[src_v.at[k]], gbuf.at[0, k], sem)
        for k in range(KR):
            pltpu.async_copy(hp_hbm.at[src_v.at[KR + k]], gbuf.at[1, k],
                             sem)

        def body(g, carry):
            gb = lax.rem(g, 3)
            base = g * KR
            for k in range(KR):
                pltpu.make_async_copy(
                    hp_hbm.at[src_v.at[base + k]],
                    gbuf.at[gb, k], sem).wait()

            @pl.when(g >= 1)
            def _():
                for k in range(KR):
                    pltpu.make_async_copy(
                        gbuf.at[lax.rem(g - 1, 3), k],
                        acc_sh.at[dst_v.at[(g - 1) * KR + k]], sem_s).wait()

            @pl.when(g + 2 < NGRP)
            def _():
                for k in range(KR):
                    pltpu.async_copy(
                        hp_hbm.at[src_v.at[(g + 2) * KR + k]],
                        gbuf.at[lax.rem(g + 2, 3), k], sem)

            for k in range(KR):
                pltpu.async_copy(gbuf.at[gb, k],
                                 acc_sh.at[dst_v.at[base + k]], sem_s,
                                 add=True)
            return carry

        lax.fori_loop(0, NGRP, body, 0)
        for k in range(KR):
            pltpu.make_async_copy(
                gbuf.at[lax.rem(NGRP - 1, 3), k],
                acc_sh.at[dst_v.at[(NGRP - 1) * KR + k]], sem_s).wait()

    @pl.when(cid == 0)
    def _():
        run(hpL_hbm)

    @pl.when(cid == 1)
    def _():
        run(hpR_hbm)

    plsc.subcore_barrier()
    for k in range(ACC_SL // CHUNK):
        kb = k % 2
        base = sid * ACC_SL + k * CHUNK
        pltpu.sync_copy(acc_sh.at[pl.ds(base, CHUNK)], gbuf.at[0, kb])
        pltpu.sync_copy(gbuf.at[0, kb], out_hbm.at[cid, pl.ds(base, CHUNK)])


_msg_call = functools.partial(
    pl.kernel,
    out_type=jax.ShapeDtypeStruct((NC, NACC, HF), jnp.float32),
    mesh=_mesh,
    scratch_types=[
        pltpu.VMEM((NCHT, CHUNK), jnp.int32),
        pltpu.VMEM((NCHT, CHUNK), jnp.int32),
        pltpu.VMEM((3, KR, CHUNK, HF), jnp.float32),
        pltpu.SemaphoreType.DMA,
        pltpu.SemaphoreType.DMA,
        pltpu.VMEM_SHARED((NACC, HF), jnp.float32),
    ],
    compiler_params=pltpu.CompilerParams(use_tc_tiling_on_sc=False),
)(_msg_body)


def _tc1_body(x_ref, w_ref, degp_ref, hpL_ref, hpR_ref, dinv_ref):
    deg = degp_ref[0, :] + degp_ref[1, :] + 1.0
    dinv = lax.rsqrt(deg)
    h = jnp.dot(x_ref[...], w_ref[...], preferred_element_type=jnp.float32)
    hp = h * dinv[:, None]
    hpL_ref[...] = hp[:, :HF]
    hpR_ref[...] = hp[:, HF:]
    dinv_ref[...] = dinv[:, None]


def _tc1_call(x, W_gc, degp):
    return pl.pallas_call(
        _tc1_body,
        grid=(GRID,),
        in_specs=[
            pl.BlockSpec((BLK, F), lambda i: (i, 0)),
            pl.BlockSpec((F, F), lambda i: (0, 0)),
            pl.BlockSpec((NC, BLK), lambda i: (0, i)),
        ],
        out_specs=[
            pl.BlockSpec((BLK, HF), lambda i: (i, 0)),
            pl.BlockSpec((BLK, HF), lambda i: (i, 0)),
            pl.BlockSpec((BLK, 1), lambda i: (i, 0)),
        ],
        out_shape=[
            jax.ShapeDtypeStruct((N, HF), jnp.float32),
            jax.ShapeDtypeStruct((N, HF), jnp.float32),
            jax.ShapeDtypeStruct((N, 1), jnp.float32),
        ],
    )(x, W_gc, degp)


def _tc2_body(accp_ref, hpL_ref, hpR_ref, dinv_ref, t_ref, bgc_ref,
              w00_ref, b00_ref, w10_ref, b10_ref,
              w01_ref, b01_ref, w11_ref, b11_ref, wps_ref, bps_ref,
              y_ref, dist_ref, ps_ref):
    acc = jnp.concatenate(
        [accp_ref[0] + hpL_ref[...], accp_ref[1] + hpR_ref[...]], axis=1)
    dist = jnp.maximum(acc * dinv_ref[...] + bgc_ref[...], 0.0)
    dist_ref[...] = dist
    y00 = jnp.maximum(
        jnp.dot(dist, w00_ref[...], preferred_element_type=jnp.float32)
        + b00_ref[...], 0.0)
    y0 = jnp.dot(y00, w01_ref[...], preferred_element_type=jnp.float32) \
        + b01_ref[...]
    y10 = jnp.maximum(
        jnp.dot(dist, w10_ref[...], preferred_element_type=jnp.float32)
        + b10_ref[...], 0.0)
    y1 = jnp.dot(y10, w11_ref[...], preferred_element_type=jnp.float32) \
        + b11_ref[...]
    y_ref[...] = jnp.where(t_ref[...] > 0, y1, y0)
    ps = jnp.dot(dist, wps_ref[...], preferred_element_type=jnp.float32) \
        + bps_ref[...]
    ps_ref[...] = jax.nn.sigmoid(ps)


def _tc2_call(accp, hpL, hpR, dinv, t2, bgc, w00, b00, w10, b10,
              w01, b01, w11, b11, wps, bps):
    def full(shape):
        nd = len(shape)
        return pl.BlockSpec(shape, lambda i, _nd=nd: (0,) * _nd)
    return pl.pallas_call(
        _tc2_body,
        grid=(GRID,),
        in_specs=[
            pl.BlockSpec((NC, BLK, HF), lambda i: (0, i, 0)),
            pl.BlockSpec((BLK, HF), lambda i: (i, 0)),
            pl.BlockSpec((BLK, HF), lambda i: (i, 0)),
            pl.BlockSpec((BLK, 1), lambda i: (i, 0)),
            pl.BlockSpec((BLK, 1), lambda i: (i, 0)),
            full((1, F)),
            full((F, F)), full((1, F)),
            full((F, F)), full((1, F)),
            full((F, 1)), full((1, 1)),
            full((F, 1)), full((1, 1)),
            full((F, 1)), full((1, 1)),
        ],
        out_specs=[
            pl.BlockSpec((BLK, 1), lambda i: (i, 0)),
            pl.BlockSpec((BLK, F), lambda i: (i, 0)),
            pl.BlockSpec((BLK, 1), lambda i: (i, 0)),
        ],
        out_shape=[
            jax.ShapeDtypeStruct((N, 1), jnp.float32),
            jax.ShapeDtypeStruct((N, F), jnp.float32),
            jax.ShapeDtypeStruct((N, 1), jnp.float32),
        ],
    )(accp, hpL, hpR, dinv, t2, bgc, w00, b00, w10, b10,
      w01, b01, w11, b11, wps, bps)


def kernel(x, edge_index, t, W_gc, b_gc, W_t00, b_t00, W_t10, b_t10,
           W_t01, b_t01, W_t11, b_t11, W_ps, b_ps):
    src = edge_index[0]
    dst = edge_index[1]
    npad = E_PAD - E
    ar = jnp.arange(npad, dtype=jnp.int32)
    src_p = jnp.concatenate([src, (ar * 7919) % N])
    dst_p = jnp.concatenate([dst, N + (ar % 64)])
    src3 = src_p.reshape(NROW, CHUNK)
    dst3 = dst_p.reshape(NROW, CHUNK)

    ones = jnp.ones((CHUNK,), jnp.float32)
    zeros1 = jnp.zeros((DEG_SL,), jnp.float32)
    zrows = jnp.zeros((CHUNK, HF), jnp.float32)

    degp = _deg_call(dst3, ones, zeros1).reshape(NC, NDEG)
    hpL, hpR, dinv = _tc1_call(x, W_gc, degp)
    accp = _msg_call(hpL, hpR, src3, dst3, zrows)
    y, dist, ps = _tc2_call(
        accp, hpL, hpR, dinv, t.reshape(N, 1),
        b_gc.reshape(1, F), W_t00, b_t00.reshape(1, F),
        W_t10, b_t10.reshape(1, F), W_t01, b_t01.reshape(1, 1),
        W_t11, b_t11.reshape(1, 1), W_ps, b_ps.reshape(1, 1))
    return (y.reshape(-1), dist, ps.reshape(-1))


# speedup vs baseline: 1.4335x; 1.0303x over previous
import functools

import jax
import jax.numpy as jnp
from jax import lax
from jax.experimental import pallas as pl
from jax.experimental.pallas import tpu as pltpu
from jax.experimental.pallas import tpu_sc as plsc

N = 10000
F = 128
E = 320000
NC = 2
NS = 16
NW = NC * NS
CHUNK = 128
NCH = 80
EPT = NCH * CHUNK
E_PAD = NW * EPT
NROW = E_PAD // CHUNK
NDEG = 10112
DEG_SL = NDEG // NS
NACC = 10240
ACC_SL = NACC // NS
HF = F // NC
NCHT = NCH * 2
KR = 2
NGRP = NCHT // KR
GROW = KR * CHUNK
BLK = 1024
GRID = (N + BLK - 1) // BLK

_mesh = plsc.VectorSubcoreMesh(
    core_axis_name="c", subcore_axis_name="s", num_cores=NC, num_subcores=NS)


def _deg_body(dst_hbm, ones_hbm, zeros_hbm, out_hbm, idx_v, ones_v, zbuf_v,
              deg_sh):
    cid = lax.axis_index("c")
    sid = lax.axis_index("s")
    wid = sid * NC + cid
    pltpu.sync_copy(zeros_hbm, zbuf_v)
    pltpu.sync_copy(zbuf_v, deg_sh.at[pl.ds(sid * DEG_SL, DEG_SL)])
    pltpu.sync_copy(ones_hbm, ones_v)
    pltpu.sync_copy(dst_hbm.at[pl.ds(wid * NCH, NCH)], idx_v)
    plsc.subcore_barrier()

    def body(j, carry):
        pltpu.sync_copy(ones_v, deg_sh.at[idx_v.at[j]], add=True)
        return carry

    lax.fori_loop(0, NCH, body, 0)
    plsc.subcore_barrier()
    pltpu.sync_copy(deg_sh.at[pl.ds(sid * DEG_SL, DEG_SL)], zbuf_v)
    pltpu.sync_copy(zbuf_v,
                    out_hbm.at[pl.ds(cid * NDEG + sid * DEG_SL, DEG_SL)])


_deg_call = functools.partial(
    pl.kernel,
    out_type=jax.ShapeDtypeStruct((NC * NDEG,), jnp.float32),
    mesh=_mesh,
    scratch_types=[
        pltpu.VMEM((NCH, CHUNK), jnp.int32),
        pltpu.VMEM((CHUNK,), jnp.float32),
        pltpu.VMEM((DEG_SL,), jnp.float32),
        pltpu.VMEM_SHARED((NDEG,), jnp.float32),
    ],
)(_deg_body)


def _msg_body(hpL_hbm, hpR_hbm, src_hbm, dst_hbm, zrows_hbm, out_hbm,
              src_v, dst_v, gbuf, sem, sem_s, acc_sh):
    cid = lax.axis_index("c")
    sid = lax.axis_index("s")
    pltpu.sync_copy(zrows_hbm, gbuf.at[0, 0])
    for k in range(ACC_SL // CHUNK):
        pltpu.sync_copy(gbuf.at[0, 0],
                        acc_sh.at[pl.ds(sid * ACC_SL + k * CHUNK, CHUNK)])
    pltpu.sync_copy(src_hbm.at[pl.ds(sid * NCHT, NCHT)], src_v)
    pltpu.sync_copy(dst_hbm.at[pl.ds(sid * NCHT, NCHT)], dst_v)
    plsc.subcore_barrier()

    def run(hp_hbm):
        for k in range(KR):
            pltpu.async_copy(hp_hbm.at[src_v.at[k]], gbuf.at[0, k], sem)
        for k in range(KR):
            pltpu.async_copy(hp_hbm.at[src_v.at[KR + k]], gbuf.at[1, k],
                             sem)

        def body(g, carry):
            gb = lax.rem(g, 3)
            base = g * KR
            for k in range(KR):
                pltpu.make_async_copy(
                    hp_hbm.at[src_v.at[base + k]],
                    gbuf.at[gb, k], sem).wait()

            @pl.when(g >= 1)
            def _():
                for k in range(KR):
                    pltpu.make_async_copy(
                        gbuf.at[lax.rem(g - 1, 3), k],
                        acc_sh.at[dst_v.at[(g - 1) * KR + k]], sem_s).wait()

            @pl.when(g + 2 < NGRP)
            def _():
                for k in range(KR):
                    pltpu.async_copy(
                        hp_hbm.at[src_v.at[(g + 2) * KR + k]],
                        gbuf.at[lax.rem(g + 2, 3), k], sem)

            for k in range(KR):
                pltpu.async_copy(gbuf.at[gb, k],
                                 acc_sh.at[dst_v.at[base + k]], sem_s,
                                 add=True)
            return carry

        lax.fori_loop(0, NGRP, body, 0)
        for k in range(KR):
            pltpu.make_async_copy(
                gbuf.at[lax.rem(NGRP - 1, 3), k],
                acc_sh.at[dst_v.at[(NGRP - 1) * KR + k]], sem_s).wait()

    @pl.when(cid == 0)
    def _():
        run(hpL_hbm)

    @pl.when(cid == 1)
    def _():
        run(hpR_hbm)

    plsc.subcore_barrier()
    for k in range(ACC_SL // CHUNK):
        kb = k % 2
        base = sid * ACC_SL + k * CHUNK
        pltpu.sync_copy(acc_sh.at[pl.ds(base, CHUNK)], gbuf.at[0, kb])
        pltpu.sync_copy(gbuf.at[0, kb], out_hbm.at[cid, pl.ds(base, CHUNK)])


_msg_call = functools.partial(
    pl.kernel,
    out_type=jax.ShapeDtypeStruct((NC, NACC, HF), jnp.float32),
    mesh=_mesh,
    scratch_types=[
        pltpu.VMEM((NCHT, CHUNK), jnp.int32),
        pltpu.VMEM((NCHT, CHUNK), jnp.int32),
        pltpu.VMEM((3, KR, CHUNK, HF), jnp.float32),
        pltpu.SemaphoreType.DMA,
        pltpu.SemaphoreType.DMA,
        pltpu.VMEM_SHARED((NACC, HF), jnp.float32),
    ],
    compiler_params=pltpu.CompilerParams(use_tc_tiling_on_sc=False),
)(_msg_body)


def _mm_body(x_ref, w_ref, h_ref):
    h_ref[...] = jnp.dot(x_ref[...], w_ref[...],
                         preferred_element_type=jnp.float32)


def _mm_call(x, W_gc):
    return pl.pallas_call(
        _mm_body,
        grid=(GRID,),
        in_specs=[
            pl.BlockSpec((BLK, F), lambda i: (i, 0)),
            pl.BlockSpec((F, F), lambda i: (0, 0)),
        ],
        out_specs=pl.BlockSpec((BLK, F), lambda i: (i, 0)),
        out_shape=jax.ShapeDtypeStruct((N, F), jnp.float32),
    )(x, W_gc)


def _tc1_body(h_ref, degp_ref, hpL_ref, hpR_ref, dinv_ref):
    deg = degp_ref[0, :] + degp_ref[1, :] + 1.0
    dinv = lax.rsqrt(deg)
    hp = h_ref[...] * dinv[:, None]
    hpL_ref[...] = hp[:, :HF]
    hpR_ref[...] = hp[:, HF:]
    dinv_ref[...] = dinv[:, None]


def _tc1_call(h, degp):
    return pl.pallas_call(
        _tc1_body,
        grid=(GRID,),
        in_specs=[
            pl.BlockSpec((BLK, F), lambda i: (i, 0)),
            pl.BlockSpec((NC, BLK), lambda i: (0, i)),
        ],
        out_specs=[
            pl.BlockSpec((BLK, HF), lambda i: (i, 0)),
            pl.BlockSpec((BLK, HF), lambda i: (i, 0)),
            pl.BlockSpec((BLK, 1), lambda i: (i, 0)),
        ],
        out_shape=[
            jax.ShapeDtypeStruct((N, HF), jnp.float32),
            jax.ShapeDtypeStruct((N, HF), jnp.float32),
            jax.ShapeDtypeStruct((N, 1), jnp.float32),
        ],
    )(h, degp)


def _tc2_body(accp_ref, hpL_ref, hpR_ref, dinv_ref, t_ref, bgc_ref,
              w00_ref, b00_ref, w10_ref, b10_ref,
              w01t_ref, b01_ref, w11t_ref, b11_ref, wpst_ref, bps_ref,
              y_ref, dist_ref, ps_ref):
    acc = jnp.concatenate(
        [accp_ref[0] + hpL_ref[...], accp_ref[1] + hpR_ref[...]], axis=1)
    dist = jnp.maximum(acc * dinv_ref[...] + bgc_ref[...], 0.0)
    dist_ref[...] = dist
    y00 = jnp.maximum(
        jnp.dot(dist, w00_ref[...], preferred_element_type=jnp.float32)
        + b00_ref[...], 0.0)
    y0 = jnp.sum(y00 * w01t_ref[...], axis=1) + b01_ref[0, 0]
    y10 = jnp.maximum(
        jnp.dot(dist, w10_ref[...], preferred_element_type=jnp.float32)
        + b10_ref[...], 0.0)
    y1 = jnp.sum(y10 * w11t_ref[...], axis=1) + b11_ref[0, 0]
    y_ref[...] = jnp.where(t_ref[...] > 0, y1, y0)
    ps = jnp.sum(dist * wpst_ref[...], axis=1) + bps_ref[0, 0]
    ps_ref[...] = jax.nn.sigmoid(ps)


def _tc2_call(accp, hpL, hpR, dinv, t, bgc, w00, b00, w10, b10,
              w01t, b01, w11t, b11, wpst, bps):
    def full(shape):
        nd = len(shape)
        return pl.BlockSpec(shape, lambda i, _nd=nd: (0,) * _nd)
    return pl.pallas_call(
        _tc2_body,
        grid=(GRID,),
        in_specs=[
            pl.BlockSpec((NC, BLK, HF), lambda i: (0, i, 0)),
            pl.BlockSpec((BLK, HF), lambda i: (i, 0)),
            pl.BlockSpec((BLK, HF), lambda i: (i, 0)),
            pl.BlockSpec((BLK, 1), lambda i: (i, 0)),
            pl.BlockSpec((BLK,), lambda i: (i,)),
            full((1, F)),
            full((F, F)), full((1, F)),
            full((F, F)), full((1, F)),
            full((1, F)), full((1, 1)),
            full((1, F)), full((1, 1)),
            full((1, F)), full((1, 1)),
        ],
        out_specs=[
            pl.BlockSpec((BLK,), lambda i: (i,)),
            pl.BlockSpec((BLK, F), lambda i: (i, 0)),
            pl.BlockSpec((BLK,), lambda i: (i,)),
        ],
        out_shape=[
            jax.ShapeDtypeStruct((N,), jnp.float32),
            jax.ShapeDtypeStruct((N, F), jnp.float32),
            jax.ShapeDtypeStruct((N,), jnp.float32),
        ],
    )(accp, hpL, hpR, dinv, t, bgc, w00, b00, w10, b10,
      w01t, b01, w11t, b11, wpst, bps)


def kernel(x, edge_index, t, W_gc, b_gc, W_t00, b_t00, W_t10, b_t10,
           W_t01, b_t01, W_t11, b_t11, W_ps, b_ps):
    src = edge_index[0]
    dst = edge_index[1]
    npad = E_PAD - E
    ar = jnp.arange(npad, dtype=jnp.int32)
    src_p = jnp.concatenate([src, (ar * 7919) % N])
    dst_p = jnp.concatenate([dst, N + (ar % 64)])
    src3 = src_p.reshape(NROW, CHUNK)
    dst3 = dst_p.reshape(NROW, CHUNK)

    ones = jnp.ones((CHUNK,), jnp.float32)
    zeros1 = jnp.zeros((DEG_SL,), jnp.float32)
    zrows = jnp.zeros((CHUNK, HF), jnp.float32)

    h = _mm_call(x, W_gc)
    degp = _deg_call(dst3, ones, zeros1).reshape(NC, NDEG)
    hpL, hpR, dinv = _tc1_call(h, degp)
    accp = _msg_call(hpL, hpR, src3, dst3, zrows)
    y, dist, ps = _tc2_call(
        accp, hpL, hpR, dinv, t,
        b_gc.reshape(1, F), W_t00, b_t00.reshape(1, F),
        W_t10, b_t10.reshape(1, F),
        W_t01.reshape(1, F), b_t01.reshape(1, 1),
        W_t11.reshape(1, F), b_t11.reshape(1, 1),
        W_ps.reshape(1, F), b_ps.reshape(1, 1))
    return (y, dist, ps)


# speedup vs baseline: 1.4710x; 1.0261x over previous
import functools

import jax
import jax.numpy as jnp
from jax import lax
from jax.experimental import pallas as pl
from jax.experimental.pallas import tpu as pltpu
from jax.experimental.pallas import tpu_sc as plsc

N = 10000
F = 128
E = 320000
NC = 2
NS = 16
NW = NC * NS
CHUNK = 128
NCH = 80
EPT = NCH * CHUNK
E_PAD = NW * EPT
NROW = E_PAD // CHUNK
NDEG = 10112
DEG_SL = NDEG // NS
NACC = 10240
ACC_SL = NACC // NS
HF = F // NC
NCHT = NCH * 2
KR = 2
NGRP = NCHT // KR
GROW = KR * CHUNK
BLK = 1024
GRID = (N + BLK - 1) // BLK

_mesh = plsc.VectorSubcoreMesh(
    core_axis_name="c", subcore_axis_name="s", num_cores=NC, num_subcores=NS)


def _deg_body(dst_hbm, ones_hbm, zeros_hbm, out_hbm, idx_v, ones_v, zbuf_v,
              semd, deg_sh):
    cid = lax.axis_index("c")
    sid = lax.axis_index("s")
    wid = sid * NC + cid
    pltpu.sync_copy(zeros_hbm, zbuf_v)
    pltpu.sync_copy(zbuf_v, deg_sh.at[pl.ds(sid * DEG_SL, DEG_SL)])
    pltpu.sync_copy(ones_hbm, ones_v)
    pltpu.sync_copy(dst_hbm.at[pl.ds(wid * NCH, NCH)], idx_v)
    plsc.subcore_barrier()

    def body(j, carry):
        for k in range(8):
            pltpu.async_copy(ones_v, deg_sh.at[idx_v.at[j * 8 + k]], semd,
                             add=True)
        for k in range(8):
            pltpu.make_async_copy(ones_v, deg_sh.at[idx_v.at[j * 8 + k]],
                                  semd).wait()
        return carry

    lax.fori_loop(0, NCH // 8, body, 0)
    plsc.subcore_barrier()
    pltpu.sync_copy(deg_sh.at[pl.ds(sid * DEG_SL, DEG_SL)], zbuf_v)
    pltpu.sync_copy(zbuf_v,
                    out_hbm.at[pl.ds(cid * NDEG + sid * DEG_SL, DEG_SL)])


_deg_call = functools.partial(
    pl.kernel,
    out_type=jax.ShapeDtypeStruct((NC * NDEG,), jnp.float32),
    mesh=_mesh,
    scratch_types=[
        pltpu.VMEM((NCH, CHUNK), jnp.int32),
        pltpu.VMEM((CHUNK,), jnp.float32),
        pltpu.VMEM((DEG_SL,), jnp.float32),
        pltpu.SemaphoreType.DMA,
        pltpu.VMEM_SHARED((NDEG,), jnp.float32),
    ],
)(_deg_body)


def _msg_body(hpL_hbm, hpR_hbm, src_hbm, dst_hbm, zrows_hbm, out_hbm,
              src_v, dst_v, gbuf, sem, sem_s, acc_sh):
    cid = lax.axis_index("c")
    sid = lax.axis_index("s")
    pltpu.sync_copy(zrows_hbm, gbuf.at[0, 0])
    for k in range(ACC_SL // CHUNK):
        pltpu.sync_copy(gbuf.at[0, 0],
                        acc_sh.at[pl.ds(sid * ACC_SL + k * CHUNK, CHUNK)])
    pltpu.sync_copy(src_hbm.at[pl.ds(sid * NCHT, NCHT)], src_v)
    pltpu.sync_copy(dst_hbm.at[pl.ds(sid * NCHT, NCHT)], dst_v)
    plsc.subcore_barrier()

    def run(hp_hbm):
        for g0 in range(2):
            for k in range(KR):
                pltpu.async_copy(hp_hbm.at[src_v.at[g0 * KR + k]],
                                 gbuf.at[g0, k], sem)

        def body(g, carry):
            gb = lax.rem(g, 3)
            base = g * KR
            for k in range(KR):
                pltpu.make_async_copy(
                    hp_hbm.at[src_v.at[base + k]],
                    gbuf.at[gb, k], sem).wait()

            @pl.when(g >= 1)
            def _():
                for k in range(KR):
                    pltpu.make_async_copy(
                        gbuf.at[lax.rem(g - 1, 3), k],
                        acc_sh.at[dst_v.at[(g - 1) * KR + k]], sem_s).wait()

            @pl.when(g + 2 < NGRP)
            def _():
                for k in range(KR):
                    pltpu.async_copy(
                        hp_hbm.at[src_v.at[(g + 2) * KR + k]],
                        gbuf.at[lax.rem(g + 2, 3), k], sem)

            for k in range(KR):
                pltpu.async_copy(gbuf.at[gb, k],
                                 acc_sh.at[dst_v.at[base + k]], sem_s,
                                 add=True)
            return carry

        lax.fori_loop(0, NGRP, body, 0)
        for k in range(KR):
            pltpu.make_async_copy(
                gbuf.at[lax.rem(NGRP - 1, 3), k],
                acc_sh.at[dst_v.at[(NGRP - 1) * KR + k]], sem_s).wait()

    @pl.when(cid == 0)
    def _():
        run(hpL_hbm)

    @pl.when(cid == 1)
    def _():
        run(hpR_hbm)

    plsc.subcore_barrier()
    for k in range(ACC_SL // CHUNK):
        kb = k % 2
        base = sid * ACC_SL + k * CHUNK
        if k >= 2:
            pbase = sid * ACC_SL + (k - 2) * CHUNK
            pltpu.make_async_copy(gbuf.at[0, kb],
                                  out_hbm.at[cid, pl.ds(pbase, CHUNK)],
                                  sem).wait()
        pltpu.sync_copy(acc_sh.at[pl.ds(base, CHUNK)], gbuf.at[0, kb])
        pltpu.async_copy(gbuf.at[0, kb], out_hbm.at[cid, pl.ds(base, CHUNK)],
                         sem)
    for k in range(ACC_SL // CHUNK - 2, ACC_SL // CHUNK):
        base = sid * ACC_SL + k * CHUNK
        pltpu.make_async_copy(gbuf.at[0, k % 2],
                              out_hbm.at[cid, pl.ds(base, CHUNK)],
                              sem).wait()


_msg_call = functools.partial(
    pl.kernel,
    out_type=jax.ShapeDtypeStruct((NC, NACC, HF), jnp.float32),
    mesh=_mesh,
    scratch_types=[
        pltpu.VMEM((NCHT, CHUNK), jnp.int32),
        pltpu.VMEM((NCHT, CHUNK), jnp.int32),
        pltpu.VMEM((3, KR, CHUNK, HF), jnp.float32),
        pltpu.SemaphoreType.DMA,
        pltpu.SemaphoreType.DMA,
        pltpu.VMEM_SHARED((NACC, HF), jnp.float32),
    ],
    compiler_params=pltpu.CompilerParams(use_tc_tiling_on_sc=False),
)(_msg_body)


def _mm_body(x_ref, w_ref, h_ref):
    h_ref[...] = jnp.dot(x_ref[...], w_ref[...],
                         preferred_element_type=jnp.float32)


def _mm_call(x, W_gc):
    return pl.pallas_call(
        _mm_body,
        grid=(GRID,),
        in_specs=[
            pl.BlockSpec((BLK, F), lambda i: (i, 0)),
            pl.BlockSpec((F, F), lambda i: (0, 0)),
        ],
        out_specs=pl.BlockSpec((BLK, F), lambda i: (i, 0)),
        out_shape=jax.ShapeDtypeStruct((N, F), jnp.float32),
    )(x, W_gc)


def _tc1_body(h_ref, degp_ref, hpL_ref, hpR_ref, dinv_ref):
    deg = degp_ref[0, :] + degp_ref[1, :] + 1.0
    dinv = lax.rsqrt(deg)
    hp = h_ref[...] * dinv[:, None]
    hpL_ref[...] = hp[:, :HF]
    hpR_ref[...] = hp[:, HF:]
    dinv_ref[...] = dinv[:, None]


def _tc1_call(h, degp):
    return pl.pallas_call(
        _tc1_body,
        grid=(GRID,),
        in_specs=[
            pl.BlockSpec((BLK, F), lambda i: (i, 0)),
            pl.BlockSpec((NC, BLK), lambda i: (0, i)),
        ],
        out_specs=[
            pl.BlockSpec((BLK, HF), lambda i: (i, 0)),
            pl.BlockSpec((BLK, HF), lambda i: (i, 0)),
            pl.BlockSpec((BLK, 1), lambda i: (i, 0)),
        ],
        out_shape=[
            jax.ShapeDtypeStruct((N, HF), jnp.float32),
            jax.ShapeDtypeStruct((N, HF), jnp.float32),
            jax.ShapeDtypeStruct((N, 1), jnp.float32),
        ],
    )(h, degp)


def _tc2_body(accp_ref, hpL_ref, hpR_ref, dinv_ref, t_ref, bgc_ref,
              w00_ref, b00_ref, w10_ref, b10_ref,
              w01t_ref, b01_ref, w11t_ref, b11_ref, wpst_ref, bps_ref,
              y_ref, dist_ref, ps_ref):
    acc = jnp.concatenate(
        [accp_ref[0] + hpL_ref[...], accp_ref[1] + hpR_ref[...]], axis=1)
    dist = jnp.maximum(acc * dinv_ref[...] + bgc_ref[...], 0.0)
    dist_ref[...] = dist
    y00 = jnp.maximum(
        jnp.dot(dist, w00_ref[...], preferred_element_type=jnp.float32)
        + b00_ref[...], 0.0)
    y0 = jnp.sum(y00 * w01t_ref[...], axis=1) + b01_ref[0, 0]
    y10 = jnp.maximum(
        jnp.dot(dist, w10_ref[...], preferred_element_type=jnp.float32)
        + b10_ref[...], 0.0)
    y1 = jnp.sum(y10 * w11t_ref[...], axis=1) + b11_ref[0, 0]
    y_ref[...] = jnp.where(t_ref[...] > 0, y1, y0)
    ps = jnp.sum(dist * wpst_ref[...], axis=1) + bps_ref[0, 0]
    ps_ref[...] = jax.nn.sigmoid(ps)


def _tc2_call(accp, hpL, hpR, dinv, t, bgc, w00, b00, w10, b10,
              w01t, b01, w11t, b11, wpst, bps):
    def full(shape):
        nd = len(shape)
        return pl.BlockSpec(shape, lambda i, _nd=nd: (0,) * _nd)
    return pl.pallas_call(
        _tc2_body,
        grid=(GRID,),
        in_specs=[
            pl.BlockSpec((NC, BLK, HF), lambda i: (0, i, 0)),
            pl.BlockSpec((BLK, HF), lambda i: (i, 0)),
            pl.BlockSpec((BLK, HF), lambda i: (i, 0)),
            pl.BlockSpec((BLK, 1), lambda i: (i, 0)),
            pl.BlockSpec((BLK,), lambda i: (i,)),
            full((1, F)),
            full((F, F)), full((1, F)),
            full((F, F)), full((1, F)),
            full((1, F)), full((1, 1)),
            full((1, F)), full((1, 1)),
            full((1, F)), full((1, 1)),
        ],
        out_specs=[
            pl.BlockSpec((BLK,), lambda i: (i,)),
            pl.BlockSpec((BLK, F), lambda i: (i, 0)),
            pl.BlockSpec((BLK,), lambda i: (i,)),
        ],
        out_shape=[
            jax.ShapeDtypeStruct((N,), jnp.float32),
            jax.ShapeDtypeStruct((N, F), jnp.float32),
            jax.ShapeDtypeStruct((N,), jnp.float32),
        ],
    )(accp, hpL, hpR, dinv, t, bgc, w00, b00, w10, b10,
      w01t, b01, w11t, b11, wpst, bps)


def kernel(x, edge_index, t, W_gc, b_gc, W_t00, b_t00, W_t10, b_t10,
           W_t01, b_t01, W_t11, b_t11, W_ps, b_ps):
    src = edge_index[0]
    dst = edge_index[1]
    npad = E_PAD - E
    ar = jnp.arange(npad, dtype=jnp.int32)
    src_p = jnp.concatenate([src, (ar * 7919) % N])
    dst_p = jnp.concatenate([dst, N + (ar % 64)])
    src3 = src_p.reshape(NROW, CHUNK)
    dst3 = dst_p.reshape(NROW, CHUNK)

    ones = jnp.ones((CHUNK,), jnp.float32)
    zeros1 = jnp.zeros((DEG_SL,), jnp.float32)
    zrows = jnp.zeros((CHUNK, HF), jnp.float32)

    h = _mm_call(x, W_gc)
    degp = _deg_call(dst3, ones, zeros1).reshape(NC, NDEG)
    hpL, hpR, dinv = _tc1_call(h, degp)
    accp = _msg_call(hpL, hpR, src3, dst3, zrows)
    y, dist, ps = _tc2_call(
        accp, hpL, hpR, dinv, t,
        b_gc.reshape(1, F), W_t00, b_t00.reshape(1, F),
        W_t10, b_t10.reshape(1, F),
        W_t01.reshape(1, F), b_t01.reshape(1, 1),
        W_t11.reshape(1, F), b_t11.reshape(1, 1),
        W_ps.reshape(1, F), b_ps.reshape(1, 1))
    return (y, dist, ps)


# speedup vs baseline: 1.5365x; 1.0445x over previous
import functools

import jax
import jax.numpy as jnp
from jax import lax
from jax.experimental import pallas as pl
from jax.experimental.pallas import tpu as pltpu
from jax.experimental.pallas import tpu_sc as plsc

N = 10000
F = 128
E = 320000
NC = 2
NS = 16
NW = NC * NS
CHUNK = 128
NROW = E // CHUNK
NROWP = 2512
NCHD = 80
NCHM = 160
NDEG = 10112
DEG_SL = NDEG // NS
NACC = 10240
ACC_SL = NACC // NS
HF = F // NC
KR = 2
BLK = 1024
GRID = (N + BLK - 1) // BLK

_mesh = plsc.VectorSubcoreMesh(
    core_axis_name="c", subcore_axis_name="s", num_cores=NC, num_subcores=NS)


def _deg_body(ei_hbm, ones_hbm, zeros_hbm, deg_out, src_out, dst_out,
              ev, src_v, dst_v, ones_v, zbuf_v, semd, semw, deg_sh):
    cid = lax.axis_index("c")
    sid = lax.axis_index("s")
    wid = sid * NC + cid
    is_last = wid == NW - 1
    nch = jnp.where(is_last, NROW - NCHD * (NW - 1), NCHD)
    ebase = wid * NCHD
    pltpu.sync_copy(zeros_hbm, zbuf_v)
    pltpu.sync_copy(zbuf_v, deg_sh.at[pl.ds(sid * DEG_SL, DEG_SL)])
    pltpu.sync_copy(ones_hbm, ones_v)

    @pl.when(jnp.logical_not(is_last))
    def _():
        pltpu.sync_copy(
            ei_hbm.at[pl.ds(0, 2), pl.ds(ebase * CHUNK, NCHD * CHUNK)],
            ev.at[pl.ds(0, 2), pl.ds(0, NCHD * CHUNK)])

    @pl.when(is_last)
    def _():
        pltpu.sync_copy(
            ei_hbm.at[pl.ds(0, 2), pl.ds(ebase * CHUNK, 20 * CHUNK)],
            ev.at[pl.ds(0, 2), pl.ds(0, 20 * CHUNK)])

    def vcopy(j, carry):
        for k in range(CHUNK // 16):
            sl = pl.ds(j * CHUNK + k * 16, 16)
            src_v[j, pl.ds(k * 16, 16)] = ev[0, sl]
            dst_v[j, pl.ds(k * 16, 16)] = ev[1, sl]
        return carry

    lax.fori_loop(0, nch, vcopy, 0)

    @pl.when(jnp.logical_not(is_last))
    def _():
        pltpu.async_copy(src_v.at[pl.ds(0, NCHD)],
                         src_out.at[pl.ds(ebase, NCHD)], semw)
        pltpu.async_copy(dst_v.at[pl.ds(0, NCHD)],
                         dst_out.at[pl.ds(ebase, NCHD)], semw)

    @pl.when(is_last)
    def _():
        pltpu.async_copy(src_v.at[pl.ds(0, 24)],
                         src_out.at[pl.ds(ebase, 24)], semw)
        pltpu.async_copy(dst_v.at[pl.ds(0, 24)],
                         dst_out.at[pl.ds(ebase, 24)], semw)

    def hbody(j, carry):
        pltpu.async_copy(ones_v, deg_sh.at[dst_v.at[j]], semd, add=True)

        @pl.when(j >= 1)
        def _():
            pltpu.make_async_copy(ones_v, deg_sh.at[dst_v.at[j - 1]],
                                  semd).wait()

        return carry

    lax.fori_loop(0, nch, hbody, 0)
    pltpu.make_async_copy(ones_v, deg_sh.at[dst_v.at[nch - 1]], semd).wait()

    @pl.when(jnp.logical_not(is_last))
    def _():
        pltpu.make_async_copy(src_v.at[pl.ds(0, NCHD)],
                              src_out.at[pl.ds(ebase, NCHD)], semw).wait()
        pltpu.make_async_copy(dst_v.at[pl.ds(0, NCHD)],
                              dst_out.at[pl.ds(ebase, NCHD)], semw).wait()

    @pl.when(is_last)
    def _():
        pltpu.make_async_copy(src_v.at[pl.ds(0, 24)],
                              src_out.at[pl.ds(ebase, 24)], semw).wait()
        pltpu.make_async_copy(dst_v.at[pl.ds(0, 24)],
                              dst_out.at[pl.ds(ebase, 24)], semw).wait()

    plsc.subcore_barrier()
    pltpu.sync_copy(deg_sh.at[pl.ds(sid * DEG_SL, DEG_SL)], zbuf_v)
    pltpu.sync_copy(zbuf_v,
                    deg_out.at[pl.ds(cid * NDEG + sid * DEG_SL, DEG_SL)])


_deg_call = functools.partial(
    pl.kernel,
    out_type=[
        jax.ShapeDtypeStruct((NC * NDEG,), jnp.float32),
        jax.ShapeDtypeStruct((NROWP, CHUNK), jnp.int32),
        jax.ShapeDtypeStruct((NROWP, CHUNK), jnp.int32),
    ],
    mesh=_mesh,
    scratch_types=[
        pltpu.VMEM((2, NCHD * CHUNK), jnp.int32),
        pltpu.VMEM((NCHD, CHUNK), jnp.int32),
        pltpu.VMEM((NCHD, CHUNK), jnp.int32),
        pltpu.VMEM((CHUNK,), jnp.float32),
        pltpu.VMEM((DEG_SL,), jnp.float32),
        pltpu.SemaphoreType.DMA,
        pltpu.SemaphoreType.DMA,
        pltpu.VMEM_SHARED((NDEG,), jnp.float32),
    ],
)(_deg_body)


def _msg_body(hpL_hbm, hpR_hbm, src_hbm, dst_hbm, zrows_hbm, out_hbm,
              src_v, dst_v, gbuf, sem, sem_s, acc_sh):
    cid = lax.axis_index("c")
    sid = lax.axis_index("s")
    pltpu.sync_copy(zrows_hbm, gbuf.at[0, 0])
    for k in range(ACC_SL // CHUNK):
        pltpu.sync_copy(gbuf.at[0, 0],
                        acc_sh.at[pl.ds(sid * ACC_SL + k * CHUNK, CHUNK)])
    rbase = jnp.where(sid < 8, sid * 160,
                      jnp.where(sid < 15, 1280 + (sid - 8) * 152, 2344))
    ngrp = jnp.where(sid < 8, 80, jnp.where(sid < 15, 76, 78))
    pltpu.sync_copy(src_hbm.at[pl.ds(rbase, NCHM)], src_v)
    pltpu.sync_copy(dst_hbm.at[pl.ds(rbase, NCHM)], dst_v)
    plsc.subcore_barrier()

    def run(hp_hbm):
        for g0 in range(2):
            for k in range(KR):
                pltpu.async_copy(hp_hbm.at[src_v.at[g0 * KR + k]],
                                 gbuf.at[g0, k], sem)

        def body(g, carry):
            gb = lax.rem(g, 3)
            base = g * KR
            for k in range(KR):
                pltpu.make_async_copy(
                    hp_hbm.at[src_v.at[base + k]],
                    gbuf.at[gb, k], sem).wait()

            @pl.when(g >= 1)
            def _():
                for k in range(KR):
                    pltpu.make_async_copy(
                        gbuf.at[lax.rem(g - 1, 3), k],
                        acc_sh.at[dst_v.at[(g - 1) * KR + k]], sem_s).wait()

            @pl.when(g + 2 < ngrp)
            def _():
                for k in range(KR):
                    pltpu.async_copy(
                        hp_hbm.at[src_v.at[(g + 2) * KR + k]],
                        gbuf.at[lax.rem(g + 2, 3), k], sem)

            for k in range(KR):
                pltpu.async_copy(gbuf.at[gb, k],
                                 acc_sh.at[dst_v.at[base + k]], sem_s,
                                 add=True)
            return carry

        lax.fori_loop(0, ngrp, body, 0)
        for k in range(KR):
            pltpu.make_async_copy(
                gbuf.at[(ngrp - 1) % 3, k],
                acc_sh.at[dst_v.at[(ngrp - 1) * KR + k]], sem_s).wait()

    @pl.when(cid == 0)
    def _():
        run(hpL_hbm)

    @pl.when(cid == 1)
    def _():
        run(hpR_hbm)

    plsc.subcore_barrier()
    for k in range(ACC_SL // CHUNK):
        kb = k % 2
        base = sid * ACC_SL + k * CHUNK
        if k >= 2:
            pbase = sid * ACC_SL + (k - 2) * CHUNK
            pltpu.make_async_copy(gbuf.at[0, kb],
                                  out_hbm.at[cid, pl.ds(pbase, CHUNK)],
                                  sem).wait()
        pltpu.sync_copy(acc_sh.at[pl.ds(base, CHUNK)], gbuf.at[0, kb])
        pltpu.async_copy(gbuf.at[0, kb], out_hbm.at[cid, pl.ds(base, CHUNK)],
                         sem)
    for k in range(ACC_SL // CHUNK - 2, ACC_SL // CHUNK):
        base = sid * ACC_SL + k * CHUNK
        pltpu.make_async_copy(gbuf.at[0, k % 2],
                              out_hbm.at[cid, pl.ds(base, CHUNK)],
                              sem).wait()


_msg_call = functools.partial(
    pl.kernel,
    out_type=jax.ShapeDtypeStruct((NC, NACC, HF), jnp.float32),
    mesh=_mesh,
    scratch_types=[
        pltpu.VMEM((NCHM, CHUNK), jnp.int32),
        pltpu.VMEM((NCHM, CHUNK), jnp.int32),
        pltpu.VMEM((3, KR, CHUNK, HF), jnp.float32),
        pltpu.SemaphoreType.DMA,
        pltpu.SemaphoreType.DMA,
        pltpu.VMEM_SHARED((NACC, HF), jnp.float32),
    ],
    compiler_params=pltpu.CompilerParams(use_tc_tiling_on_sc=False),
)(_msg_body)


def _mm_body(x_ref, w_ref, h_ref):
    h_ref[...] = jnp.dot(x_ref[...], w_ref[...],
                         preferred_element_type=jnp.float32)


def _mm_call(x, W_gc):
    return pl.pallas_call(
        _mm_body,
        grid=(GRID,),
        in_specs=[
            pl.BlockSpec((BLK, F), lambda i: (i, 0)),
            pl.BlockSpec((F, F), lambda i: (0, 0)),
        ],
        out_specs=pl.BlockSpec((BLK, F), lambda i: (i, 0)),
        out_shape=jax.ShapeDtypeStruct((N, F), jnp.float32),
    )(x, W_gc)


def _tc1_body(h_ref, degp_ref, hpL_ref, hpR_ref, dinv_ref):
    deg = degp_ref[0, :] + degp_ref[1, :] + 1.0
    dinv = lax.rsqrt(deg)
    hp = h_ref[...] * dinv[:, None]
    hpL_ref[...] = hp[:, :HF]
    hpR_ref[...] = hp[:, HF:]
    dinv_ref[...] = dinv[:, None]


def _tc1_call(h, degp):
    return pl.pallas_call(
        _tc1_body,
        grid=(GRID,),
        in_specs=[
            pl.BlockSpec((BLK, F), lambda i: (i, 0)),
            pl.BlockSpec((NC, BLK), lambda i: (0, i)),
        ],
        out_specs=[
            pl.BlockSpec((BLK, HF), lambda i: (i, 0)),
            pl.BlockSpec((BLK, HF), lambda i: (i, 0)),
            pl.BlockSpec((BLK, 1), lambda i: (i, 0)),
        ],
        out_shape=[
            jax.ShapeDtypeStruct((N, HF), jnp.float32),
            jax.ShapeDtypeStruct((N, HF), jnp.float32),
            jax.ShapeDtypeStruct((N, 1), jnp.float32),
        ],
    )(h, degp)


def _tc2_body(accp_ref, hpL_ref, hpR_ref, dinv_ref, t_ref, bgc_ref,
              w00_ref, b00_ref, w10_ref, b10_ref,
              w01t_ref, b01_ref, w11t_ref, b11_ref, wpst_ref, bps_ref,
              y_ref, dist_ref, ps_ref):
    acc = jnp.concatenate(
        [accp_ref[0] + hpL_ref[...], accp_ref[1] + hpR_ref[...]], axis=1)
    dist = jnp.maximum(acc * dinv_ref[...] + bgc_ref[...], 0.0)
    dist_ref[...] = dist
    y00 = jnp.maximum(
        jnp.dot(dist, w00_ref[...], preferred_element_type=jnp.float32)
        + b00_ref[...], 0.0)
    y0 = jnp.sum(y00 * w01t_ref[...], axis=1) + b01_ref[0, 0]
    y10 = jnp.maximum(
        jnp.dot(dist, w10_ref[...], preferred_element_type=jnp.float32)
        + b10_ref[...], 0.0)
    y1 = jnp.sum(y10 * w11t_ref[...], axis=1) + b11_ref[0, 0]
    y_ref[...] = jnp.where(t_ref[...] > 0, y1, y0)
    ps = jnp.sum(dist * wpst_ref[...], axis=1) + bps_ref[0, 0]
    ps_ref[...] = jax.nn.sigmoid(ps)


def _tc2_call(accp, hpL, hpR, dinv, t, bgc, w00, b00, w10, b10,
              w01t, b01, w11t, b11, wpst, bps):
    def full(shape):
        nd = len(shape)
        return pl.BlockSpec(shape, lambda i, _nd=nd: (0,) * _nd)
    return pl.pallas_call(
        _tc2_body,
        grid=(GRID,),
        in_specs=[
            pl.BlockSpec((NC, BLK, HF), lambda i: (0, i, 0)),
            pl.BlockSpec((BLK, HF), lambda i: (i, 0)),
            pl.BlockSpec((BLK, HF), lambda i: (i, 0)),
            pl.BlockSpec((BLK, 1), lambda i: (i, 0)),
            pl.BlockSpec((BLK,), lambda i: (i,)),
            full((1, F)),
            full((F, F)), full((1, F)),
            full((F, F)), full((1, F)),
            full((1, F)), full((1, 1)),
            full((1, F)), full((1, 1)),
            full((1, F)), full((1, 1)),
        ],
        out_specs=[
            pl.BlockSpec((BLK,), lambda i: (i,)),
            pl.BlockSpec((BLK, F), lambda i: (i, 0)),
            pl.BlockSpec((BLK,), lambda i: (i,)),
        ],
        out_shape=[
            jax.ShapeDtypeStruct((N,), jnp.float32),
            jax.ShapeDtypeStruct((N, F), jnp.float32),
            jax.ShapeDtypeStruct((N,), jnp.float32),
        ],
    )(accp, hpL, hpR, dinv, t, bgc, w00, b00, w10, b10,
      w01t, b01, w11t, b11, wpst, bps)


def kernel(x, edge_index, t, W_gc, b_gc, W_t00, b_t00, W_t10, b_t10,
           W_t01, b_t01, W_t11, b_t11, W_ps, b_ps):
    ones = jnp.ones((CHUNK,), jnp.float32)
    zeros1 = jnp.zeros((DEG_SL,), jnp.float32)
    zrows = jnp.zeros((CHUNK, HF), jnp.float32)

    h = _mm_call(x, W_gc)
    degp, src3, dst3 = _deg_call(edge_index, ones, zeros1)
    degp = degp.reshape(NC, NDEG)
    hpL, hpR, dinv = _tc1_call(h, degp)
    accp = _msg_call(hpL, hpR, src3, dst3, zrows)
    y, dist, ps = _tc2_call(
        accp, hpL, hpR, dinv, t,
        b_gc.reshape(1, F), W_t00, b_t00.reshape(1, F),
        W_t10, b_t10.reshape(1, F),
        W_t01.reshape(1, F), b_t01.reshape(1, 1),
        W_t11.reshape(1, F), b_t11.reshape(1, 1),
        W_ps.reshape(1, F), b_ps.reshape(1, 1))
    return (y, dist, ps)
